# Initial kernel scaffold; baseline (speedup 1.0000x reference)
import jax
import jax.numpy as jnp
from jax.experimental import pallas as pl


def kernel(x, neighborhood):
    raise NotImplementedError("write your pallas kernel here")


# speedup vs baseline: 2.3189x; 2.3189x over previous
import functools

import jax
import jax.numpy as jnp
from jax import lax
from jax.experimental import pallas as pl
from jax.experimental.pallas import tpu as pltpu
from jax.experimental.pallas import tpu_sc as plsc

_NUM_CORES = 2
_NUM_SUBCORES = 16
_NUM_WORKERS = _NUM_CORES * _NUM_SUBCORES
_ROWS_PER_CHUNK = 8
_BLOCK_OUT = 512
_GROUP = 16


@functools.cache
def _build(n_rows, n_in, n_out_p, n_neigh):
  assert n_rows % (_NUM_WORKERS * _ROWS_PER_CHUNK) == 0
  assert n_out_p % _BLOCK_OUT == 0
  rounds = n_rows // (_NUM_WORKERS * _ROWS_PER_CHUNK)
  n_blocks = n_out_p // _BLOCK_OUT
  groups = _BLOCK_OUT // _GROUP

  mesh = plsc.VectorSubcoreMesh(
      core_axis_name="c", subcore_axis_name="s",
      num_cores=_NUM_CORES, num_subcores=_NUM_SUBCORES)

  @functools.partial(
      pl.kernel,
      mesh=mesh,
      out_type=jax.ShapeDtypeStruct((n_rows, n_out_p), jnp.float32),
      scratch_types=[
          pltpu.VMEM((_ROWS_PER_CHUNK, n_in), jnp.float32),
          pltpu.VMEM((n_neigh, _BLOCK_OUT), jnp.int32),
          pltpu.VMEM((_ROWS_PER_CHUNK, _BLOCK_OUT), jnp.float32),
      ],
      compiler_params=pltpu.CompilerParams(
          use_tc_tiling_on_sc=False, needs_layout_passes=False),
  )
  def k(x2_hbm, nbt_hbm, out_hbm, slab, nb, ost):
    wid = lax.axis_index("s") * _NUM_CORES + lax.axis_index("c")

    def round_body(r, carry):
      rc = wid * rounds + r
      row0 = rc * _ROWS_PER_CHUNK
      pltpu.sync_copy(x2_hbm.at[pl.ds(row0, _ROWS_PER_CHUNK), :], slab)

      def block_body(ob, carry):
        col0 = ob * _BLOCK_OUT
        pltpu.sync_copy(nbt_hbm.at[:, pl.ds(col0, _BLOCK_OUT)], nb)

        def group_body(og, carry):
          g0 = og * _GROUP
          idxs = [nb[j, pl.ds(g0, _GROUP)] for j in range(n_neigh)]
          for f in range(_ROWS_PER_CHUNK):
            rowv = jnp.full((_GROUP,), f, jnp.int32)
            m = plsc.load_gather(slab, [rowv, idxs[0]])
            for j in range(1, n_neigh):
              m = jnp.maximum(m, plsc.load_gather(slab, [rowv, idxs[j]]))
            ost[f, pl.ds(g0, _GROUP)] = m
          return carry

        lax.fori_loop(0, groups, group_body, 0)
        pltpu.sync_copy(
            ost,
            out_hbm.at[pl.ds(row0, _ROWS_PER_CHUNK),
                       pl.ds(col0, _BLOCK_OUT)])
        return carry

      lax.fori_loop(0, n_blocks, block_body, 0)
      return carry

    lax.fori_loop(0, rounds, round_body, 0)

  return k


def kernel(x, neighborhood):
  b, f, n_in = x.shape
  n_out, n_neigh = neighborhood.shape
  n_rows = b * f

  x2 = x.reshape(n_rows, n_in)
  n_out_p = -(-n_out // _BLOCK_OUT) * _BLOCK_OUT
  idx = neighborhood.astype(jnp.int32)
  if n_out_p > n_out:
    idx = jnp.concatenate(
        [idx, jnp.zeros((n_out_p - n_out, n_neigh), jnp.int32)], axis=0)
  nbt = idx.T

  out2 = _build(n_rows, n_in, n_out_p, n_neigh)(x2, nbt)
  return out2[:, :n_out].reshape(b, f, n_out)
